# baseline (speedup 1.0000x reference)
import functools

import jax
import jax.numpy as jnp
from jax import lax
from jax.experimental import pallas as pl
from jax.experimental.pallas import tpu as pltpu
from jax.experimental.pallas import tpu_sc as plsc

EMBED_DIM = 1024
NBINS = 6
BATCH = 16384

_info = plsc.get_sparse_core_info()
_NC, _NS = _info.num_cores, _info.num_subcores
_NW = _NC * _NS
_B_PER_W = BATCH // _NW
_GRP = 16
_N_GRPS = _B_PER_W // _GRP

_TC_BLK = 1024


def _sc_embed1(idx0, tab_flat):
    mesh = plsc.VectorSubcoreMesh(core_axis_name="c", subcore_axis_name="s")
    flat = jax.ShapeDtypeStruct((BATCH * EMBED_DIM,), jnp.float32)

    @functools.partial(
        pl.kernel,
        out_type=flat,
        mesh=mesh,
        scratch_types=[
            pltpu.VMEM((_B_PER_W,), jnp.int32),
            pltpu.VMEM((NBINS * EMBED_DIM,), jnp.float32),
            pltpu.SemaphoreType.DMA,
        ],
    )
    def body(idx_hbm, tab_hbm, out_hbm, idx_v, tab_v, ssem):
        wid = lax.axis_index("s") * _NC + lax.axis_index("c")
        base = wid * _B_PER_W

        pltpu.sync_copy(tab_hbm, tab_v)
        pltpu.sync_copy(idx_hbm.at[pl.ds(base, _B_PER_W)], idx_v)

        def drain_group():
            for _ in range(_GRP):
                pltpu.make_async_copy(
                    tab_v.at[pl.ds(0, EMBED_DIM)],
                    out_hbm.at[pl.ds(0, EMBED_DIM)],
                    ssem,
                ).wait()

        @pl.loop(0, _N_GRPS)
        def _grp(g):
            vec = idx_v[pl.ds(g * _GRP, _GRP)]
            for j in range(_GRP):
                src = vec[j] * EMBED_DIM
                dst = (base + g * _GRP + j) * EMBED_DIM
                pltpu.async_copy(
                    tab_v.at[pl.ds(src, EMBED_DIM)],
                    out_hbm.at[pl.ds(dst, EMBED_DIM)],
                    ssem,
                )

            @pl.when(g > 0)
            def _lagged_drain():
                drain_group()

        drain_group()

    return body(idx0, tab_flat)


def _tc_body(i1_ref, i2_ref, t1_ref, t2_ref, o1_ref, o2_ref):
    for iref, tref, oref in ((i1_ref, t1_ref, o1_ref), (i2_ref, t2_ref, o2_ref)):
        idx = iref[...]
        oh = (
            idx[:, None] == lax.broadcasted_iota(jnp.int32, (_TC_BLK, 8), 1)
        ).astype(jnp.float32)
        oref[...] = jnp.dot(
            oh,
            tref[...],
            preferred_element_type=jnp.float32,
            precision=lax.Precision.HIGHEST,
        )


def _tc_embed2(idx1, idx2, tab1_pad, tab2_pad):
    out = jax.ShapeDtypeStruct((BATCH, EMBED_DIM), jnp.float32)
    return pl.pallas_call(
        _tc_body,
        grid=(BATCH // _TC_BLK,),
        in_specs=[
            pl.BlockSpec((_TC_BLK,), lambda i: (i,)),
            pl.BlockSpec((_TC_BLK,), lambda i: (i,)),
            pl.BlockSpec((8, EMBED_DIM), lambda i: (0, 0)),
            pl.BlockSpec((8, EMBED_DIM), lambda i: (0, 0)),
        ],
        out_specs=[
            pl.BlockSpec((_TC_BLK, EMBED_DIM), lambda i: (i, 0)),
            pl.BlockSpec((_TC_BLK, EMBED_DIM), lambda i: (i, 0)),
        ],
        out_shape=[out, out],
        compiler_params=pltpu.CompilerParams(
            dimension_semantics=("arbitrary",),
        ),
    )(idx1, idx2, tab1_pad, tab2_pad)


def kernel(x, helix_table, sheet_table, turns_table):
    xi = x.astype(jnp.int32)
    pad = jnp.zeros((8 - NBINS, EMBED_DIM), jnp.float32)
    o0 = _sc_embed1(xi[:, 0], helix_table.reshape(-1))
    o1, o2 = _tc_embed2(
        xi[:, 1],
        xi[:, 2],
        jnp.concatenate([sheet_table, pad], axis=0),
        jnp.concatenate([turns_table, pad], axis=0),
    )
    return (o0.reshape(BATCH, EMBED_DIM), o1, o2)


# speedup vs baseline: 1.0001x; 1.0001x over previous
import functools

import jax
import jax.numpy as jnp
from jax import lax
from jax.experimental import pallas as pl
from jax.experimental.pallas import tpu as pltpu
from jax.experimental.pallas import tpu_sc as plsc

EMBED_DIM = 1024
NBINS = 6
BATCH = 16384

_info = plsc.get_sparse_core_info()
_NC, _NS = _info.num_cores, _info.num_subcores
_NW = _NC * _NS
_B_PER_W = BATCH // _NW
_CHUNK = 16
_NBUF = 2
_N_CHUNKS = _B_PER_W // _CHUNK
_CHUNK_ELEMS = _CHUNK * EMBED_DIM

_TC_BLK = 1024


def _sc_embed1(idx0, tab_flat):
    mesh = plsc.VectorSubcoreMesh(core_axis_name="c", subcore_axis_name="s")
    flat = jax.ShapeDtypeStruct((BATCH * EMBED_DIM,), jnp.float32)

    @functools.partial(
        pl.kernel,
        out_type=flat,
        mesh=mesh,
        scratch_types=[
            pltpu.VMEM((_B_PER_W,), jnp.int32),
            pltpu.VMEM((NBINS * EMBED_DIM,), jnp.float32),
            [pltpu.VMEM((_CHUNK_ELEMS,), jnp.float32) for _ in range(_NBUF)],
            [pltpu.SemaphoreType.DMA for _ in range(_NBUF)],
        ],
    )
    def body(idx_hbm, tab_hbm, out_hbm, idx_v, tab_v, stage, ssem):
        wid = lax.axis_index("s") * _NC + lax.axis_index("c")
        base = wid * _B_PER_W

        pltpu.sync_copy(tab_hbm, tab_v)
        pltpu.sync_copy(idx_hbm.at[pl.ds(base, _B_PER_W)], idx_v)

        def compute(n, b):
            vec = idx_v[pl.ds(n * _CHUNK, _CHUNK)]
            rb = [vec[j] * EMBED_DIM for j in range(_CHUNK)]

            @plsc.parallel_loop(0, EMBED_DIM // 16, unroll=4)
            def _col(c):
                coff = c * 16
                for j in range(_CHUNK):
                    stage[b][pl.ds(j * EMBED_DIM + coff, 16)] = tab_v[
                        pl.ds(rb[j] + coff, 16)
                    ]

        def scatter(n, b):
            off = (base + n * _CHUNK) * EMBED_DIM
            pltpu.async_copy(
                stage[b], out_hbm.at[pl.ds(off, _CHUNK_ELEMS)], ssem[b]
            )

        def scatter_wait(b):
            pltpu.make_async_copy(
                stage[b], out_hbm.at[pl.ds(0, _CHUNK_ELEMS)], ssem[b]
            ).wait()

        @pl.loop(0, _N_CHUNKS, step=_NBUF)
        def _steady(j):
            for b in range(_NBUF):

                @pl.when(j > 0)
                def _drain():
                    scatter_wait(b)

                compute(j + b, b)
                scatter(j + b, b)

        for b in range(_NBUF):
            scatter_wait(b)

    return body(idx0, tab_flat)


def _tc_body(i1_ref, i2_ref, t1_ref, t2_ref, o1_ref, o2_ref):
    for iref, tref, oref in ((i1_ref, t1_ref, o1_ref), (i2_ref, t2_ref, o2_ref)):
        idx = iref[...]
        oh = (
            idx[:, None] == lax.broadcasted_iota(jnp.int32, (_TC_BLK, 8), 1)
        ).astype(jnp.float32)
        oref[...] = jnp.dot(
            oh,
            tref[...],
            preferred_element_type=jnp.float32,
            precision=lax.Precision.HIGHEST,
        )


def _tc_embed2(idx1, idx2, tab1_pad, tab2_pad):
    out = jax.ShapeDtypeStruct((BATCH, EMBED_DIM), jnp.float32)
    return pl.pallas_call(
        _tc_body,
        grid=(BATCH // _TC_BLK,),
        in_specs=[
            pl.BlockSpec((_TC_BLK,), lambda i: (i,)),
            pl.BlockSpec((_TC_BLK,), lambda i: (i,)),
            pl.BlockSpec((8, EMBED_DIM), lambda i: (0, 0)),
            pl.BlockSpec((8, EMBED_DIM), lambda i: (0, 0)),
        ],
        out_specs=[
            pl.BlockSpec((_TC_BLK, EMBED_DIM), lambda i: (i, 0)),
            pl.BlockSpec((_TC_BLK, EMBED_DIM), lambda i: (i, 0)),
        ],
        out_shape=[out, out],
        compiler_params=pltpu.CompilerParams(
            dimension_semantics=("arbitrary",),
        ),
    )(idx1, idx2, tab1_pad, tab2_pad)


def kernel(x, helix_table, sheet_table, turns_table):
    xi = x.astype(jnp.int32)
    pad = jnp.zeros((8 - NBINS, EMBED_DIM), jnp.float32)
    o0 = _sc_embed1(xi[:, 0], helix_table.reshape(-1))
    o1, o2 = _tc_embed2(
        xi[:, 1],
        xi[:, 2],
        jnp.concatenate([sheet_table, pad], axis=0),
        jnp.concatenate([turns_table, pad], axis=0),
    )
    return (o0.reshape(BATCH, EMBED_DIM), o1, o2)


# speedup vs baseline: 1.3163x; 1.3161x over previous
import functools

import jax
import jax.numpy as jnp
from jax import lax
from jax.experimental import pallas as pl
from jax.experimental.pallas import tpu as pltpu
from jax.experimental.pallas import tpu_sc as plsc

EMBED_DIM = 1024
NBINS = 6
BATCH = 16384

_info = plsc.get_sparse_core_info()
_NC, _NS = _info.num_cores, _info.num_subcores
_NW = _NC * _NS
_B_PER_W = BATCH // _NW
_CHUNK = 16
_NBUF = 2
_N_CHUNKS = _B_PER_W // _CHUNK
_CHUNK_ELEMS = _CHUNK * EMBED_DIM

_TC_BLK = 1024


def _sc_embed1(idx0, tab_flat):
    mesh = plsc.VectorSubcoreMesh(core_axis_name="c", subcore_axis_name="s")
    flat = jax.ShapeDtypeStruct((BATCH * EMBED_DIM,), jnp.float32)

    @functools.partial(
        pl.kernel,
        out_type=flat,
        mesh=mesh,
        scratch_types=[
            pltpu.VMEM((_B_PER_W,), jnp.int32),
            pltpu.VMEM((NBINS * EMBED_DIM,), jnp.float32),
            [pltpu.VMEM((_CHUNK_ELEMS,), jnp.float32) for _ in range(_NBUF)],
            [pltpu.SemaphoreType.DMA for _ in range(_NBUF)],
        ],
    )
    def body(idx_hbm, tab_hbm, out_hbm, idx_v, tab_v, stage, ssem):
        wid = lax.axis_index("s") * _NC + lax.axis_index("c")
        base = wid * _B_PER_W

        pltpu.sync_copy(tab_hbm, tab_v)
        pltpu.sync_copy(idx_hbm.at[pl.ds(base, _B_PER_W)], idx_v)

        def compute(n, b):
            vec = idx_v[pl.ds(n * _CHUNK, _CHUNK)]
            rb = [vec[j] * EMBED_DIM for j in range(_CHUNK)]

            @plsc.parallel_loop(0, EMBED_DIM // 16, unroll=4)
            def _col(c):
                coff = c * 16
                for j in range(_CHUNK):
                    stage[b][pl.ds(j * EMBED_DIM + coff, 16)] = tab_v[
                        pl.ds(rb[j] + coff, 16)
                    ]

        def scatter(n, b):
            off = (base + n * _CHUNK) * EMBED_DIM
            pltpu.async_copy(
                stage[b], out_hbm.at[pl.ds(off, _CHUNK_ELEMS)], ssem[b]
            )

        def scatter_wait(b):
            pltpu.make_async_copy(
                stage[b], out_hbm.at[pl.ds(0, _CHUNK_ELEMS)], ssem[b]
            ).wait()

        @pl.loop(0, _N_CHUNKS, step=_NBUF)
        def _steady(j):
            for b in range(_NBUF):

                @pl.when(j > 0)
                def _drain():
                    scatter_wait(b)

                compute(j + b, b)
                scatter(j + b, b)

        for b in range(_NBUF):
            scatter_wait(b)

    return body(idx0, tab_flat)


def _tc_body(i1_ref, i2_ref, t1_ref, t2_ref, o1_ref, o2_ref):
    for iref, tref, oref in ((i1_ref, t1_ref, o1_ref), (i2_ref, t2_ref, o2_ref)):
        idx = iref[...]
        oh = (
            idx[:, None] == lax.broadcasted_iota(jnp.int32, (_TC_BLK, 8), 1)
        ).astype(jnp.float32)
        oref[...] = jnp.dot(oh, tref[...], preferred_element_type=jnp.float32)


def _tc_embed2(idx1, idx2, tab1_pad, tab2_pad):
    out = jax.ShapeDtypeStruct((BATCH, EMBED_DIM), jnp.float32)
    return pl.pallas_call(
        _tc_body,
        grid=(BATCH // _TC_BLK,),
        in_specs=[
            pl.BlockSpec((_TC_BLK,), lambda i: (i,)),
            pl.BlockSpec((_TC_BLK,), lambda i: (i,)),
            pl.BlockSpec((8, EMBED_DIM), lambda i: (0, 0)),
            pl.BlockSpec((8, EMBED_DIM), lambda i: (0, 0)),
        ],
        out_specs=[
            pl.BlockSpec((_TC_BLK, EMBED_DIM), lambda i: (i, 0)),
            pl.BlockSpec((_TC_BLK, EMBED_DIM), lambda i: (i, 0)),
        ],
        out_shape=[out, out],
        compiler_params=pltpu.CompilerParams(
            dimension_semantics=("arbitrary",),
        ),
    )(idx1, idx2, tab1_pad, tab2_pad)


def kernel(x, helix_table, sheet_table, turns_table):
    xi = x.astype(jnp.int32)
    pad = jnp.zeros((8 - NBINS, EMBED_DIM), jnp.float32)
    o0 = _sc_embed1(xi[:, 0], helix_table.reshape(-1))
    o1, o2 = _tc_embed2(
        xi[:, 1],
        xi[:, 2],
        jnp.concatenate([sheet_table, pad], axis=0),
        jnp.concatenate([turns_table, pad], axis=0),
    )
    return (o0.reshape(BATCH, EMBED_DIM), o1, o2)


# speedup vs baseline: 1.3232x; 1.0052x over previous
import functools

import jax
import jax.numpy as jnp
from jax import lax
from jax.experimental import pallas as pl
from jax.experimental.pallas import tpu as pltpu
from jax.experimental.pallas import tpu_sc as plsc

EMBED_DIM = 1024
NBINS = 6
BATCH = 16384

_info = plsc.get_sparse_core_info()
_NC, _NS = _info.num_cores, _info.num_subcores
_NW = _NC * _NS
_B_PER_W = BATCH // _NW
_CHUNK = 16
_NBUF = 2
_N_CHUNKS = _B_PER_W // _CHUNK
_CHUNK_ELEMS = _CHUNK * EMBED_DIM

_TC_BLK = 512


def _sc_embed1(idx0, tab_flat):
    mesh = plsc.VectorSubcoreMesh(core_axis_name="c", subcore_axis_name="s")
    flat = jax.ShapeDtypeStruct((BATCH * EMBED_DIM,), jnp.float32)

    @functools.partial(
        pl.kernel,
        out_type=flat,
        mesh=mesh,
        scratch_types=[
            pltpu.VMEM((_B_PER_W,), jnp.int32),
            pltpu.VMEM((NBINS * EMBED_DIM,), jnp.float32),
            [pltpu.VMEM((_CHUNK_ELEMS,), jnp.float32) for _ in range(_NBUF)],
            [pltpu.SemaphoreType.DMA for _ in range(_NBUF)],
        ],
    )
    def body(idx_hbm, tab_hbm, out_hbm, idx_v, tab_v, stage, ssem):
        wid = lax.axis_index("s") * _NC + lax.axis_index("c")
        base = wid * _B_PER_W

        pltpu.sync_copy(tab_hbm, tab_v)
        pltpu.sync_copy(idx_hbm.at[pl.ds(base, _B_PER_W)], idx_v)

        def compute(n, b):
            vec = idx_v[pl.ds(n * _CHUNK, _CHUNK)]
            rb = [vec[j] * EMBED_DIM for j in range(_CHUNK)]

            @plsc.parallel_loop(0, EMBED_DIM // 16, unroll=4)
            def _col(c):
                coff = c * 16
                for j in range(_CHUNK):
                    stage[b][pl.ds(j * EMBED_DIM + coff, 16)] = tab_v[
                        pl.ds(rb[j] + coff, 16)
                    ]

        def scatter(n, b):
            off = (base + n * _CHUNK) * EMBED_DIM
            pltpu.async_copy(
                stage[b], out_hbm.at[pl.ds(off, _CHUNK_ELEMS)], ssem[b]
            )

        def scatter_wait(b):
            pltpu.make_async_copy(
                stage[b], out_hbm.at[pl.ds(0, _CHUNK_ELEMS)], ssem[b]
            ).wait()

        @pl.loop(0, _N_CHUNKS, step=_NBUF)
        def _steady(j):
            for b in range(_NBUF):

                @pl.when(j > 0)
                def _drain():
                    scatter_wait(b)

                compute(j + b, b)
                scatter(j + b, b)

        for b in range(_NBUF):
            scatter_wait(b)

    return body(idx0, tab_flat)


def _tc_body(i1_ref, i2_ref, t1_ref, t2_ref, o1_ref, o2_ref):
    for iref, tref, oref in ((i1_ref, t1_ref, o1_ref), (i2_ref, t2_ref, o2_ref)):
        idx = iref[...]
        oh = (
            idx[:, None] == lax.broadcasted_iota(jnp.int32, (_TC_BLK, 8), 1)
        ).astype(jnp.float32)
        oref[...] = jnp.dot(oh, tref[...], preferred_element_type=jnp.float32)


def _tc_embed2(idx1, idx2, tab1_pad, tab2_pad):
    out = jax.ShapeDtypeStruct((BATCH, EMBED_DIM), jnp.float32)
    return pl.pallas_call(
        _tc_body,
        grid=(BATCH // _TC_BLK,),
        in_specs=[
            pl.BlockSpec((_TC_BLK,), lambda i: (i,)),
            pl.BlockSpec((_TC_BLK,), lambda i: (i,)),
            pl.BlockSpec((8, EMBED_DIM), lambda i: (0, 0)),
            pl.BlockSpec((8, EMBED_DIM), lambda i: (0, 0)),
        ],
        out_specs=[
            pl.BlockSpec((_TC_BLK, EMBED_DIM), lambda i: (i, 0)),
            pl.BlockSpec((_TC_BLK, EMBED_DIM), lambda i: (i, 0)),
        ],
        out_shape=[out, out],
        compiler_params=pltpu.CompilerParams(
            dimension_semantics=("arbitrary",),
        ),
    )(idx1, idx2, tab1_pad, tab2_pad)


def kernel(x, helix_table, sheet_table, turns_table):
    xi = x.astype(jnp.int32)
    pad = jnp.zeros((8 - NBINS, EMBED_DIM), jnp.float32)
    o0 = _sc_embed1(xi[:, 0], helix_table.reshape(-1))
    o1, o2 = _tc_embed2(
        xi[:, 1],
        xi[:, 2],
        jnp.concatenate([sheet_table, pad], axis=0),
        jnp.concatenate([turns_table, pad], axis=0),
    )
    return (o0.reshape(BATCH, EMBED_DIM), o1, o2)
